# baseline (speedup 1.0000x reference)
import functools
import math

import jax
import jax.numpy as jnp
from jax import lax
from jax.experimental import pallas as pl
from jax.experimental.pallas import tpu as pltpu
from jax.experimental.pallas import tpu_sc as plsc

N = 10000
E = 320000
G = 16
NH = 256
P = 196
PD = 768
DIM = 512
HEADS = 8
DH = 64
OF = 256

_BN_SCALE = 1.0 / math.sqrt(1.0 + 1e-5)


_CW = 128
_CHUNKS = 158
_EPAD = _CHUNKS * _CW - E // 16
_DPAD = 1 << 20
_NHALF = 5120
_JROWS = 256
_ZROWS = (_NHALF + _JROWS) // 16
_NPAD = _NHALF + _JROWS


@functools.lru_cache(maxsize=None)
def _make_segsum():
  mesh = plsc.VectorSubcoreMesh(core_axis_name="c", subcore_axis_name="s")

  @functools.partial(
      pl.kernel,
      out_type=jax.ShapeDtypeStruct((2 * N, 128), jnp.float32),
      mesh=mesh,
      scratch_types=[
          pltpu.VMEM((_CHUNKS, _CW), jnp.int32),
          pltpu.VMEM((_CHUNKS, _CW), jnp.int32),
          pltpu.VMEM((_CW, 128), jnp.float32),
          pltpu.VMEM((_CW, 128), jnp.float32),
          pltpu.VMEM_SHARED((_NPAD, 128), jnp.float32),
          pltpu.SemaphoreType.DMA,
          pltpu.SemaphoreType.DMA,
      ],
  )
  def segsum(gxs, srcs2, dst2, zeros, out,
             src_v, dstp_v, rows_a, rows_b, acc, sem_a, sem_b):
    c = lax.axis_index("c")
    s = lax.axis_index("s")
    pltpu.sync_copy(srcs2.at[c * 16 + s], src_v)

    for p in (0, 1):
      lo = p * _NHALF
      hi = lo + _NHALF

      pltpu.sync_copy(dst2.at[s], dstp_v)

      def xform(r, carry):
        for k in range(_CW // 16):
          v = dstp_v[r, pl.ds(k * 16, 16)]
          inr = (v >= lo) & (v < hi)
          junk = _NHALF + (v & (_JROWS - 1))
          dstp_v[r, pl.ds(k * 16, 16)] = jnp.where(inr, v - lo, junk)
        return carry

      lax.fori_loop(0, _CHUNKS, xform, 0)

      pltpu.sync_copy(zeros, acc.at[pl.ds(s * _ZROWS, _ZROWS)])
      plsc.subcore_barrier()

      pltpu.async_copy(gxs.at[src_v.at[0]], rows_a, sem_a)

      def step(jj, carry):
        j = 2 * jj
        pltpu.make_async_copy(gxs.at[src_v.at[j]], rows_a, sem_a).wait()
        pltpu.async_copy(gxs.at[src_v.at[j + 1]], rows_b, sem_b)
        pltpu.sync_copy(rows_a, acc.at[dstp_v.at[j]], add=True)
        pltpu.make_async_copy(gxs.at[src_v.at[j + 1]], rows_b, sem_b).wait()

        @pl.when(jj < _CHUNKS // 2 - 1)
        def _():
          pltpu.async_copy(gxs.at[src_v.at[j + 2]], rows_a, sem_a)

        pltpu.sync_copy(rows_b, acc.at[dstp_v.at[j + 1]], add=True)
        return carry

      lax.fori_loop(0, _CHUNKS // 2, step, 0)
      plsc.subcore_barrier()

      if p == 0:
        pltpu.sync_copy(
            acc.at[pl.ds(pl.multiple_of(s * 320, 8), 320)],
            out.at[pl.ds(pl.multiple_of(c * N + s * 320, 8), 320)])
      else:
        @pl.when(s < 15)
        def _():
          pltpu.sync_copy(
              acc.at[pl.ds(pl.multiple_of(s * 304, 8), 304)],
              out.at[pl.ds(pl.multiple_of(c * N + _NHALF + s * 304, 8),
                           304)])

        @pl.when(s == 15)
        def _():
          pltpu.sync_copy(
              acc.at[pl.ds(15 * 304, 320)],
              out.at[pl.ds(pl.multiple_of(c * N + _NHALF + 15 * 304, 8),
                           320)])

      plsc.subcore_barrier()

  return segsum


_RB = 2000
_NBLK = N // _RB


def _gin_body(gx_ref, agglo_ref, agghi_ref, b_ref, w1_ref, b1_ref,
              w2_ref, b2_ref, bng_ref, bnb_ref, out_ref, pool_ref):
  i = pl.program_id(0)
  agg = jnp.concatenate([agglo_ref[...], agghi_ref[...]], axis=1)
  h = gx_ref[...] + agg
  z = jnp.maximum(
      jnp.dot(h, w1_ref[...], preferred_element_type=jnp.float32)
      + b1_ref[...], 0.0)
  z = jnp.dot(z, w2_ref[...], preferred_element_type=jnp.float32) + b2_ref[...]
  act = jnp.maximum(z * (bng_ref[...] * _BN_SCALE) + bnb_ref[...], 0.0)
  out_ref[...] = act

  b = b_ref[...]
  maxs = []
  sums = []
  for gg in range(G):
    m = b == gg
    maxs.append(jnp.max(jnp.where(m, act, -jnp.inf), axis=0).reshape(1, NH))
    sums.append(jnp.sum(jnp.where(m, act, 0.0), axis=0).reshape(1, NH))
  pmax = jnp.concatenate(maxs, axis=0)
  psum = jnp.concatenate(sums, axis=0)

  @pl.when(i == 0)
  def _():
    pool_ref[...] = jnp.concatenate(
        [jnp.full((G, NH), -jnp.inf, jnp.float32),
         jnp.zeros((G, NH), jnp.float32)], axis=1)

  cur = pool_ref[...]
  pool_ref[...] = jnp.concatenate(
      [jnp.maximum(cur[:, :NH], pmax), cur[:, NH:] + psum], axis=1)


def _gin_layer(gx, agg, batch2, w1, b1, w2, b2, bng, bnb):
  din = gx.shape[1]
  grid = (_NBLK,)
  return pl.pallas_call(
      _gin_body,
      grid=grid,
      in_specs=[
          pl.BlockSpec((_RB, din), lambda i: (i, 0)),
          pl.BlockSpec((_RB, 128), lambda i: (i, 0)),
          pl.BlockSpec((_RB, 128), lambda i: (i + _NBLK, 0)),
          pl.BlockSpec((_RB, 1), lambda i: (i, 0)),
          pl.BlockSpec((din, NH), lambda i: (0, 0)),
          pl.BlockSpec((1, NH), lambda i: (0, 0)),
          pl.BlockSpec((NH, NH), lambda i: (0, 0)),
          pl.BlockSpec((1, NH), lambda i: (0, 0)),
          pl.BlockSpec((1, NH), lambda i: (0, 0)),
          pl.BlockSpec((1, NH), lambda i: (0, 0)),
      ],
      out_specs=[
          pl.BlockSpec((_RB, NH), lambda i: (i, 0)),
          pl.BlockSpec((G, 2 * NH), lambda i: (0, 0)),
      ],
      out_shape=[
          jax.ShapeDtypeStruct((N, NH), jnp.float32),
          jax.ShapeDtypeStruct((G, 2 * NH), jnp.float32),
      ],
      compiler_params=pltpu.CompilerParams(
          dimension_semantics=("arbitrary",)),
  )(gx, agg, agg, batch2, w1, b1, w2, b2, bng, bnb)


def _ln_in(x, g, b):
  m = jnp.mean(x, axis=-1, keepdims=True)
  v = jnp.mean((x - m) ** 2, axis=-1, keepdims=True)
  return (x - m) * lax.rsqrt(v + 1e-5) * g + b


def _vit_body(pi_ref, wp_ref, bp_ref, ln1g_ref, ln1b_ref, wqkv_ref, bqkv_ref,
              wo_ref, bo_ref, ln2g_ref, ln2b_ref, wm1_ref, bm1_ref,
              wm2_ref, bm2_ref, wout_ref, bout_ref, bng_ref, bnb_ref,
              out_ref):
  xp = pi_ref[0]
  x = jnp.dot(xp, wp_ref[...], preferred_element_type=jnp.float32) + bp_ref[...]
  h = _ln_in(x, ln1g_ref[...], ln1b_ref[...])
  qkv = (jnp.dot(h, wqkv_ref[...], preferred_element_type=jnp.float32)
         + bqkv_ref[...])
  inv = 1.0 / math.sqrt(float(DH))
  outs = []
  for hd in range(HEADS):
    q = qkv[:, hd * DH:(hd + 1) * DH]
    k = qkv[:, DIM + hd * DH:DIM + (hd + 1) * DH]
    v = qkv[:, 2 * DIM + hd * DH:2 * DIM + (hd + 1) * DH]
    s = lax.dot_general(q, k, (((1,), (1,)), ((), ())),
                        preferred_element_type=jnp.float32) * inv
    s = s - jnp.max(s, axis=-1, keepdims=True)
    e = jnp.exp(s)
    a = e / jnp.sum(e, axis=-1, keepdims=True)
    outs.append(jnp.dot(a, v, preferred_element_type=jnp.float32))
  o = jnp.concatenate(outs, axis=1)
  x = x + jnp.dot(o, wo_ref[...], preferred_element_type=jnp.float32) + bo_ref[...]
  h = _ln_in(x, ln2g_ref[...], ln2b_ref[...])
  m = jnp.maximum(
      jnp.dot(h, wm1_ref[...], preferred_element_type=jnp.float32)
      + bm1_ref[...], 0.0)
  x = x + jnp.dot(m, wm2_ref[...], preferred_element_type=jnp.float32) + bm2_ref[...]
  pooled = jnp.mean(x, axis=0, keepdims=True)
  r = (jnp.dot(pooled, wout_ref[...], preferred_element_type=jnp.float32)
       + bout_ref[...])
  out_ref[...] = (r * (bng_ref[...] * _BN_SCALE) + bnb_ref[...]).reshape(
      1, 1, OF)


def _vit(pi, wp, bp, ln1g, ln1b, wqkv, bqkv, wo, bo, ln2g, ln2b,
         wm1, bm1, wm2, bm2, wout, bout, bng, bnb):
  full = lambda shape: pl.BlockSpec(shape, lambda g: tuple(0 for _ in shape))
  return pl.pallas_call(
      _vit_body,
      grid=(G,),
      in_specs=[
          pl.BlockSpec((1, P, PD), lambda g: (g, 0, 0)),
          full((PD, DIM)), full((1, DIM)), full((1, DIM)), full((1, DIM)),
          full((DIM, 3 * DIM)), full((1, 3 * DIM)),
          full((DIM, DIM)), full((1, DIM)), full((1, DIM)), full((1, DIM)),
          full((DIM, 256)), full((1, 256)), full((256, DIM)), full((1, DIM)),
          full((DIM, OF)), full((1, OF)), full((1, OF)), full((1, OF)),
      ],
      out_specs=pl.BlockSpec((1, 1, OF), lambda g: (g, 0, 0)),
      out_shape=jax.ShapeDtypeStruct((G, 1, OF), jnp.float32),
      compiler_params=pltpu.CompilerParams(
          dimension_semantics=("arbitrary",)),
  )(pi, wp, bp, ln1g, ln1b, wqkv, bqkv, wo, bo, ln2g, ln2b,
    wm1, bm1, wm2, bm2, wout, bout, bng, bnb).reshape(G, OF)


def _head_body(img_ref, p1_ref, p2_ref, p3_ref, p4_ref, bt_ref,
               l1w_ref, l1b_ref, l2w_ref, l2b_ref, gbg_ref, gbb_ref,
               f1w1_ref, f1b1_ref, f1w2_ref, f1b2_ref,
               f2w1_ref, f2b1_ref, f2w2_ref, f2b2_ref,
               hw_ref, hb_ref, out_ref):
  bt = bt_ref[...]
  cnts = []
  for gg in range(G):
    cnts.append(jnp.sum(jnp.where(bt == gg, 1.0, 0.0)).reshape(1, 1))
  cnt = jnp.maximum(jnp.concatenate(cnts, axis=0), 1.0)

  acc = None
  for p_ref in (p1_ref, p2_ref, p3_ref, p4_ref):
    p = p_ref[...]
    rep = jnp.concatenate([p[:, :NH], p[:, NH:] / cnt], axis=1)
    acc = rep if acc is None else acc + rep
  r1 = jnp.maximum(
      jnp.dot(acc, l1w_ref[...], preferred_element_type=jnp.float32)
      + l1b_ref[...], 0.0)
  go = (jnp.dot(r1, l2w_ref[...], preferred_element_type=jnp.float32)
        + l2b_ref[...]) * (gbg_ref[...] * _BN_SCALE) + gbb_ref[...]
  f = jnp.concatenate([img_ref[...], go], axis=1)
  f = jnp.maximum(jnp.dot(f, f1w1_ref[...], preferred_element_type=jnp.float32)
                  + f1b1_ref[...], 0.0)
  f = jnp.maximum(jnp.dot(f, f1w2_ref[...], preferred_element_type=jnp.float32)
                  + f1b2_ref[...], 0.0)
  f = jnp.maximum(jnp.dot(f, f2w1_ref[...], preferred_element_type=jnp.float32)
                  + f2b1_ref[...], 0.0)
  f = jnp.maximum(jnp.dot(f, f2w2_ref[...], preferred_element_type=jnp.float32)
                  + f2b2_ref[...], 0.0)
  out_ref[...] = (jnp.dot(f, hw_ref[...], preferred_element_type=jnp.float32)
                  + hb_ref[...])


def _head(img, p1, p2, p3, p4, bt, l1w, l1b, l2w, l2b, gbg, gbb,
          f1w1, f1b1, f1w2, f1b2, f2w1, f2b1, f2w2, f2b2, hw, hb):
  return pl.pallas_call(
      _head_body,
      out_shape=jax.ShapeDtypeStruct((G, hw.shape[1]), jnp.float32),
  )(img, p1, p2, p3, p4, bt, l1w, l1b, l2w, l2b, gbg, gbb,
    f1w1, f1b1, f1w2, f1b2, f2w1, f2b1, f2w2, f2b2, hw, hb)


def kernel(x, edge_index, edge_attr, patch_img, batch, vit_Wp, vit_bp,
           vit_ln1_g, vit_ln1_b, vit_Wqkv, vit_bqkv, vit_Wo, vit_bo,
           vit_ln2_g, vit_ln2_b, vit_Wm1, vit_bm1, vit_Wm2, vit_bm2,
           vit_Wout, vit_bout, vit_bn_g, vit_bn_b,
           g1_W1, g1_b1, g1_W2, g1_b2, bn1_g, bn1_b,
           g2_W1, g2_b1, g2_W2, g2_b2, bn2_g, bn2_b,
           g3_W1, g3_b1, g3_W2, g3_b2, bn3_g, bn3_b,
           g4_W1, g4_b1, g4_W2, g4_b2, bn4_g, bn4_b,
           lin1_W, lin1_b, lin2_W, lin2_b, ginbn_g, ginbn_b,
           f1_W1, f1_b1, f1_W2, f1_b2, f2_W1, f2_b1, f2_W2, f2_b2,
           head_W, head_b):
  del edge_attr
  r2 = lambda v: v.reshape(1, -1)

  src = jnp.concatenate(
      [edge_index[0].reshape(16, E // 16),
       jnp.zeros((16, _EPAD), jnp.int32)], axis=1).reshape(16, _CHUNKS, _CW)
  srcs2 = jnp.concatenate([src, src + N], axis=0)
  dst2 = jnp.concatenate(
      [edge_index[1].reshape(16, E // 16),
       jnp.full((16, _EPAD), _DPAD, jnp.int32)],
      axis=1).reshape(16, _CHUNKS, _CW)
  batch2 = batch.reshape(N, 1)
  bt = batch.reshape(100, 100)
  zeros128 = jnp.zeros((_ZROWS, 128), jnp.float32)

  img = _vit(patch_img, vit_Wp, r2(vit_bp), r2(vit_ln1_g), r2(vit_ln1_b),
             vit_Wqkv, r2(vit_bqkv), vit_Wo, r2(vit_bo), r2(vit_ln2_g),
             r2(vit_ln2_b), vit_Wm1, r2(vit_bm1), vit_Wm2, r2(vit_bm2),
             vit_Wout, r2(vit_bout), r2(vit_bn_g), r2(vit_bn_b))

  x_pad = jnp.concatenate([x, jnp.zeros((N, 128), jnp.float32)], axis=1)
  w1s = jnp.stack([
      jnp.concatenate([g1_W1, jnp.zeros((128, NH), jnp.float32)], axis=0),
      g2_W1, g3_W1, g4_W1])
  w2s = jnp.stack([g1_W2, g2_W2, g3_W2, g4_W2])
  b1s = jnp.stack([r2(g1_b1), r2(g2_b1), r2(g3_b1), r2(g4_b1)])
  b2s = jnp.stack([r2(g1_b2), r2(g2_b2), r2(g3_b2), r2(g4_b2)])
  bgs = jnp.stack([r2(bn1_g), r2(bn2_g), r2(bn3_g), r2(bn4_g)])
  bbs = jnp.stack([r2(bn1_b), r2(bn2_b), r2(bn3_b), r2(bn4_b)])

  def gin_step(l, carry):
    gx, pools = carry
    gxs = jnp.concatenate([gx[:, :128], gx[:, 128:]], axis=0)
    agg = _make_segsum()(gxs, srcs2, dst2, zeros128)
    gxn, pool = _gin_layer(gx, agg, batch2, w1s[l], b1s[l], w2s[l], b2s[l],
                           bgs[l], bbs[l])
    pools = lax.dynamic_update_slice(pools, pool[None], (l, 0, 0))
    return gxn, pools

  _, pools = lax.fori_loop(0, 4, gin_step,
                           (x_pad, jnp.zeros((4, G, 2 * NH), jnp.float32)))

  y = _head(img, pools[0], pools[1], pools[2], pools[3], bt,
            lin1_W, r2(lin1_b), lin2_W, r2(lin2_b), r2(ginbn_g), r2(ginbn_b),
            f1_W1, r2(f1_b1), f1_W2, r2(f1_b2), f2_W1, r2(f2_b1),
            f2_W2, r2(f2_b2), head_W, r2(head_b))
  return y


# speedup vs baseline: 1.2288x; 1.2288x over previous
import functools
import math

import jax
import jax.numpy as jnp
from jax import lax
from jax.experimental import pallas as pl
from jax.experimental.pallas import tpu as pltpu
from jax.experimental.pallas import tpu_sc as plsc

N = 10000
E = 320000
G = 16
NH = 256
P = 196
PD = 768
DIM = 512
HEADS = 8
DH = 64
OF = 256

_BN_SCALE = 1.0 / math.sqrt(1.0 + 1e-5)


_CW = 80
_BCH = 32
_CHUNKS = 256
_NBLK_IDX = _CHUNKS // _BCH
_EPAD = _CHUNKS * _CW - E // 16
_DPAD = 10040
_NPAD = 10048


@functools.lru_cache(maxsize=None)
def _make_segsum():
  mesh = plsc.VectorSubcoreMesh(core_axis_name="c", subcore_axis_name="s")

  @functools.partial(
      pl.kernel,
      out_type=jax.ShapeDtypeStruct((2 * N, 128), jnp.float32),
      mesh=mesh,
      scratch_types=[
          pltpu.VMEM((_BCH, _CW), jnp.int32),
          pltpu.VMEM((_BCH, _CW), jnp.int32),
          pltpu.VMEM((_BCH, _CW), jnp.int32),
          pltpu.VMEM((_BCH, _CW), jnp.int32),
          pltpu.VMEM((_CW, 128), jnp.float32),
          pltpu.VMEM((_CW, 128), jnp.float32),
          pltpu.VMEM_SHARED((_NPAD, 128), jnp.float32),
          pltpu.SemaphoreType.DMA,
          pltpu.SemaphoreType.DMA,
          pltpu.SemaphoreType.DMA,
          pltpu.SemaphoreType.DMA,
      ],
  )
  def segsum(gxs, srcs2, dst2, zeros, out,
             sa, da, sb, db, rows_a, rows_b, acc,
             sem_ia, sem_ib, sem_ra, sem_rb):
    c = lax.axis_index("c")
    s = lax.axis_index("s")
    w8 = (c * 16 + s) * _NBLK_IDX
    d8 = s * _NBLK_IDX

    @pl.when(s < 15)
    def _():
      pltpu.sync_copy(zeros.at[pl.ds(0, 624)],
                      acc.at[pl.ds(pl.multiple_of(s * 624, 8), 624)])

    @pl.when(s == 15)
    def _():
      pltpu.sync_copy(zeros, acc.at[pl.ds(15 * 624, 688)])

    pltpu.async_copy(srcs2.at[w8], sa, sem_ia)
    pltpu.async_copy(dst2.at[d8], da, sem_ia)
    plsc.subcore_barrier()

    def inner(sv, dv, last):
      def step(jj, carry):
        j = 2 * jj
        pltpu.make_async_copy(gxs.at[sv.at[j]], rows_a, sem_ra).wait()
        pltpu.async_copy(gxs.at[sv.at[j + 1]], rows_b, sem_rb)
        pltpu.sync_copy(rows_a, acc.at[dv.at[j]], add=True)
        pltpu.make_async_copy(gxs.at[sv.at[j + 1]], rows_b, sem_rb).wait()

        @pl.when(jj < _BCH // 2 - 1)
        def _():
          pltpu.async_copy(gxs.at[sv.at[j + 2]], rows_a, sem_ra)

        pltpu.sync_copy(rows_b, acc.at[dv.at[j + 1]], add=True)
        return carry

      pltpu.async_copy(gxs.at[sv.at[0]], rows_a, sem_ra)
      lax.fori_loop(0, _BCH // 2, step, 0)

    def outer(bb, carry):
      b0 = 2 * bb
      pltpu.make_async_copy(srcs2.at[w8 + b0], sa, sem_ia).wait()
      pltpu.make_async_copy(dst2.at[d8 + b0], da, sem_ia).wait()
      pltpu.async_copy(srcs2.at[w8 + b0 + 1], sb, sem_ib)
      pltpu.async_copy(dst2.at[d8 + b0 + 1], db, sem_ib)
      inner(sa, da, False)
      pltpu.make_async_copy(srcs2.at[w8 + b0 + 1], sb, sem_ib).wait()
      pltpu.make_async_copy(dst2.at[d8 + b0 + 1], db, sem_ib).wait()

      @pl.when(bb < _NBLK_IDX // 2 - 1)
      def _():
        pltpu.async_copy(srcs2.at[w8 + b0 + 2], sa, sem_ia)
        pltpu.async_copy(dst2.at[d8 + b0 + 2], da, sem_ia)

      inner(sb, db, bb == _NBLK_IDX // 2 - 1)
      return carry

    lax.fori_loop(0, _NBLK_IDX // 2, outer, 0)
    plsc.subcore_barrier()

    @pl.when(s < 15)
    def _():
      pltpu.sync_copy(
          acc.at[pl.ds(pl.multiple_of(s * 624, 8), 624)],
          out.at[pl.ds(pl.multiple_of(c * N + s * 624, 8), 624)])

    @pl.when(s == 15)
    def _():
      pltpu.sync_copy(
          acc.at[pl.ds(15 * 624, 640)],
          out.at[pl.ds(pl.multiple_of(c * N + 15 * 624, 8), 640)])

  return segsum


_RB = 2000
_NBLK = N // _RB


def _gin_body(gx_ref, agglo_ref, agghi_ref, b_ref, w1_ref, b1_ref,
              w2_ref, b2_ref, bng_ref, bnb_ref, out_ref, pool_ref):
  i = pl.program_id(0)
  agg = jnp.concatenate([agglo_ref[...], agghi_ref[...]], axis=1)
  h = gx_ref[...] + agg
  z = jnp.maximum(
      jnp.dot(h, w1_ref[...], preferred_element_type=jnp.float32)
      + b1_ref[...], 0.0)
  z = jnp.dot(z, w2_ref[...], preferred_element_type=jnp.float32) + b2_ref[...]
  act = jnp.maximum(z * (bng_ref[...] * _BN_SCALE) + bnb_ref[...], 0.0)
  out_ref[...] = act

  b = b_ref[...]
  maxs = []
  sums = []
  for gg in range(G):
    m = b == gg
    maxs.append(jnp.max(jnp.where(m, act, -jnp.inf), axis=0).reshape(1, NH))
    sums.append(jnp.sum(jnp.where(m, act, 0.0), axis=0).reshape(1, NH))
  pmax = jnp.concatenate(maxs, axis=0)
  psum = jnp.concatenate(sums, axis=0)

  @pl.when(i == 0)
  def _():
    pool_ref[...] = jnp.concatenate(
        [jnp.full((G, NH), -jnp.inf, jnp.float32),
         jnp.zeros((G, NH), jnp.float32)], axis=1)

  cur = pool_ref[...]
  pool_ref[...] = jnp.concatenate(
      [jnp.maximum(cur[:, :NH], pmax), cur[:, NH:] + psum], axis=1)


def _gin_layer(gx, agg, batch2, w1, b1, w2, b2, bng, bnb):
  din = gx.shape[1]
  grid = (_NBLK,)
  return pl.pallas_call(
      _gin_body,
      grid=grid,
      in_specs=[
          pl.BlockSpec((_RB, din), lambda i: (i, 0)),
          pl.BlockSpec((_RB, 128), lambda i: (i, 0)),
          pl.BlockSpec((_RB, 128), lambda i: (i + _NBLK, 0)),
          pl.BlockSpec((_RB, 1), lambda i: (i, 0)),
          pl.BlockSpec((din, NH), lambda i: (0, 0)),
          pl.BlockSpec((1, NH), lambda i: (0, 0)),
          pl.BlockSpec((NH, NH), lambda i: (0, 0)),
          pl.BlockSpec((1, NH), lambda i: (0, 0)),
          pl.BlockSpec((1, NH), lambda i: (0, 0)),
          pl.BlockSpec((1, NH), lambda i: (0, 0)),
      ],
      out_specs=[
          pl.BlockSpec((_RB, NH), lambda i: (i, 0)),
          pl.BlockSpec((G, 2 * NH), lambda i: (0, 0)),
      ],
      out_shape=[
          jax.ShapeDtypeStruct((N, NH), jnp.float32),
          jax.ShapeDtypeStruct((G, 2 * NH), jnp.float32),
      ],
      compiler_params=pltpu.CompilerParams(
          dimension_semantics=("arbitrary",)),
  )(gx, agg, agg, batch2, w1, b1, w2, b2, bng, bnb)


def _ln_in(x, g, b):
  m = jnp.mean(x, axis=-1, keepdims=True)
  v = jnp.mean((x - m) ** 2, axis=-1, keepdims=True)
  return (x - m) * lax.rsqrt(v + 1e-5) * g + b


def _vit_body(pi_ref, wp_ref, bp_ref, ln1g_ref, ln1b_ref, wqkv_ref, bqkv_ref,
              wo_ref, bo_ref, ln2g_ref, ln2b_ref, wm1_ref, bm1_ref,
              wm2_ref, bm2_ref, wout_ref, bout_ref, bng_ref, bnb_ref,
              out_ref):
  xp = pi_ref[0]
  x = jnp.dot(xp, wp_ref[...], preferred_element_type=jnp.float32) + bp_ref[...]
  h = _ln_in(x, ln1g_ref[...], ln1b_ref[...])
  qkv = (jnp.dot(h, wqkv_ref[...], preferred_element_type=jnp.float32)
         + bqkv_ref[...])
  inv = 1.0 / math.sqrt(float(DH))
  outs = []
  for hd in range(HEADS):
    q = qkv[:, hd * DH:(hd + 1) * DH]
    k = qkv[:, DIM + hd * DH:DIM + (hd + 1) * DH]
    v = qkv[:, 2 * DIM + hd * DH:2 * DIM + (hd + 1) * DH]
    s = lax.dot_general(q, k, (((1,), (1,)), ((), ())),
                        preferred_element_type=jnp.float32) * inv
    s = s - jnp.max(s, axis=-1, keepdims=True)
    e = jnp.exp(s)
    a = e / jnp.sum(e, axis=-1, keepdims=True)
    outs.append(jnp.dot(a, v, preferred_element_type=jnp.float32))
  o = jnp.concatenate(outs, axis=1)
  x = x + jnp.dot(o, wo_ref[...], preferred_element_type=jnp.float32) + bo_ref[...]
  h = _ln_in(x, ln2g_ref[...], ln2b_ref[...])
  m = jnp.maximum(
      jnp.dot(h, wm1_ref[...], preferred_element_type=jnp.float32)
      + bm1_ref[...], 0.0)
  x = x + jnp.dot(m, wm2_ref[...], preferred_element_type=jnp.float32) + bm2_ref[...]
  pooled = jnp.mean(x, axis=0, keepdims=True)
  r = (jnp.dot(pooled, wout_ref[...], preferred_element_type=jnp.float32)
       + bout_ref[...])
  out_ref[...] = (r * (bng_ref[...] * _BN_SCALE) + bnb_ref[...]).reshape(
      1, 1, OF)


def _vit(pi, wp, bp, ln1g, ln1b, wqkv, bqkv, wo, bo, ln2g, ln2b,
         wm1, bm1, wm2, bm2, wout, bout, bng, bnb):
  full = lambda shape: pl.BlockSpec(shape, lambda g: tuple(0 for _ in shape))
  return pl.pallas_call(
      _vit_body,
      grid=(G,),
      in_specs=[
          pl.BlockSpec((1, P, PD), lambda g: (g, 0, 0)),
          full((PD, DIM)), full((1, DIM)), full((1, DIM)), full((1, DIM)),
          full((DIM, 3 * DIM)), full((1, 3 * DIM)),
          full((DIM, DIM)), full((1, DIM)), full((1, DIM)), full((1, DIM)),
          full((DIM, 256)), full((1, 256)), full((256, DIM)), full((1, DIM)),
          full((DIM, OF)), full((1, OF)), full((1, OF)), full((1, OF)),
      ],
      out_specs=pl.BlockSpec((1, 1, OF), lambda g: (g, 0, 0)),
      out_shape=jax.ShapeDtypeStruct((G, 1, OF), jnp.float32),
      compiler_params=pltpu.CompilerParams(
          dimension_semantics=("arbitrary",)),
  )(pi, wp, bp, ln1g, ln1b, wqkv, bqkv, wo, bo, ln2g, ln2b,
    wm1, bm1, wm2, bm2, wout, bout, bng, bnb).reshape(G, OF)


def _head_body(img_ref, p1_ref, p2_ref, p3_ref, p4_ref, bt_ref,
               l1w_ref, l1b_ref, l2w_ref, l2b_ref, gbg_ref, gbb_ref,
               f1w1_ref, f1b1_ref, f1w2_ref, f1b2_ref,
               f2w1_ref, f2b1_ref, f2w2_ref, f2b2_ref,
               hw_ref, hb_ref, out_ref):
  bt = bt_ref[...]
  cnts = []
  for gg in range(G):
    cnts.append(jnp.sum(jnp.where(bt == gg, 1.0, 0.0)).reshape(1, 1))
  cnt = jnp.maximum(jnp.concatenate(cnts, axis=0), 1.0)

  acc = None
  for p_ref in (p1_ref, p2_ref, p3_ref, p4_ref):
    p = p_ref[...]
    rep = jnp.concatenate([p[:, :NH], p[:, NH:] / cnt], axis=1)
    acc = rep if acc is None else acc + rep
  r1 = jnp.maximum(
      jnp.dot(acc, l1w_ref[...], preferred_element_type=jnp.float32)
      + l1b_ref[...], 0.0)
  go = (jnp.dot(r1, l2w_ref[...], preferred_element_type=jnp.float32)
        + l2b_ref[...]) * (gbg_ref[...] * _BN_SCALE) + gbb_ref[...]
  f = jnp.concatenate([img_ref[...], go], axis=1)
  f = jnp.maximum(jnp.dot(f, f1w1_ref[...], preferred_element_type=jnp.float32)
                  + f1b1_ref[...], 0.0)
  f = jnp.maximum(jnp.dot(f, f1w2_ref[...], preferred_element_type=jnp.float32)
                  + f1b2_ref[...], 0.0)
  f = jnp.maximum(jnp.dot(f, f2w1_ref[...], preferred_element_type=jnp.float32)
                  + f2b1_ref[...], 0.0)
  f = jnp.maximum(jnp.dot(f, f2w2_ref[...], preferred_element_type=jnp.float32)
                  + f2b2_ref[...], 0.0)
  out_ref[...] = (jnp.dot(f, hw_ref[...], preferred_element_type=jnp.float32)
                  + hb_ref[...])


def _head(img, p1, p2, p3, p4, bt, l1w, l1b, l2w, l2b, gbg, gbb,
          f1w1, f1b1, f1w2, f1b2, f2w1, f2b1, f2w2, f2b2, hw, hb):
  return pl.pallas_call(
      _head_body,
      out_shape=jax.ShapeDtypeStruct((G, hw.shape[1]), jnp.float32),
  )(img, p1, p2, p3, p4, bt, l1w, l1b, l2w, l2b, gbg, gbb,
    f1w1, f1b1, f1w2, f1b2, f2w1, f2b1, f2w2, f2b2, hw, hb)


def kernel(x, edge_index, edge_attr, patch_img, batch, vit_Wp, vit_bp,
           vit_ln1_g, vit_ln1_b, vit_Wqkv, vit_bqkv, vit_Wo, vit_bo,
           vit_ln2_g, vit_ln2_b, vit_Wm1, vit_bm1, vit_Wm2, vit_bm2,
           vit_Wout, vit_bout, vit_bn_g, vit_bn_b,
           g1_W1, g1_b1, g1_W2, g1_b2, bn1_g, bn1_b,
           g2_W1, g2_b1, g2_W2, g2_b2, bn2_g, bn2_b,
           g3_W1, g3_b1, g3_W2, g3_b2, bn3_g, bn3_b,
           g4_W1, g4_b1, g4_W2, g4_b2, bn4_g, bn4_b,
           lin1_W, lin1_b, lin2_W, lin2_b, ginbn_g, ginbn_b,
           f1_W1, f1_b1, f1_W2, f1_b2, f2_W1, f2_b1, f2_W2, f2_b2,
           head_W, head_b):
  del edge_attr
  r2 = lambda v: v.reshape(1, -1)

  src = jnp.concatenate(
      [edge_index[0].reshape(16, E // 16),
       jnp.zeros((16, _EPAD), jnp.int32)],
      axis=1).reshape(16 * _NBLK_IDX, _BCH, _CW)
  srcs2 = jnp.concatenate([src, src + N], axis=0)
  dst2 = jnp.concatenate(
      [edge_index[1].reshape(16, E // 16),
       jnp.full((16, _EPAD), _DPAD, jnp.int32)],
      axis=1).reshape(16 * _NBLK_IDX, _BCH, _CW)
  batch2 = batch.reshape(N, 1)
  bt = batch.reshape(100, 100)
  zeros128 = jnp.zeros((688, 128), jnp.float32)

  img = _vit(patch_img, vit_Wp, r2(vit_bp), r2(vit_ln1_g), r2(vit_ln1_b),
             vit_Wqkv, r2(vit_bqkv), vit_Wo, r2(vit_bo), r2(vit_ln2_g),
             r2(vit_ln2_b), vit_Wm1, r2(vit_bm1), vit_Wm2, r2(vit_bm2),
             vit_Wout, r2(vit_bout), r2(vit_bn_g), r2(vit_bn_b))

  x_pad = jnp.concatenate([x, jnp.zeros((N, 128), jnp.float32)], axis=1)
  w1s = jnp.stack([
      jnp.concatenate([g1_W1, jnp.zeros((128, NH), jnp.float32)], axis=0),
      g2_W1, g3_W1, g4_W1])
  w2s = jnp.stack([g1_W2, g2_W2, g3_W2, g4_W2])
  b1s = jnp.stack([r2(g1_b1), r2(g2_b1), r2(g3_b1), r2(g4_b1)])
  b2s = jnp.stack([r2(g1_b2), r2(g2_b2), r2(g3_b2), r2(g4_b2)])
  bgs = jnp.stack([r2(bn1_g), r2(bn2_g), r2(bn3_g), r2(bn4_g)])
  bbs = jnp.stack([r2(bn1_b), r2(bn2_b), r2(bn3_b), r2(bn4_b)])

  def gin_step(l, carry):
    gx, pools = carry
    gxs = jnp.concatenate([gx[:, :128], gx[:, 128:]], axis=0)
    agg = _make_segsum()(gxs, srcs2, dst2, zeros128)
    gxn, pool = _gin_layer(gx, agg, batch2, w1s[l], b1s[l], w2s[l], b2s[l],
                           bgs[l], bbs[l])
    pools = lax.dynamic_update_slice(pools, pool[None], (l, 0, 0))
    return gxn, pools

  _, pools = lax.fori_loop(0, 4, gin_step,
                           (x_pad, jnp.zeros((4, G, 2 * NH), jnp.float32)))

  y = _head(img, pools[0], pools[1], pools[2], pools[3], bt,
            lin1_W, r2(lin1_b), lin2_W, r2(lin2_b), r2(ginbn_g), r2(ginbn_b),
            f1_W1, r2(f1_b1), f1_W2, r2(f1_b2), f2_W1, r2(f2_b1),
            f2_W2, r2(f2_b2), head_W, r2(head_b))
  return y


# speedup vs baseline: 1.3713x; 1.1159x over previous
import functools
import math

import jax
import jax.numpy as jnp
from jax import lax
from jax.experimental import pallas as pl
from jax.experimental.pallas import tpu as pltpu
from jax.experimental.pallas import tpu_sc as plsc

N = 10000
E = 320000
G = 16
NH = 256
P = 196
PD = 768
DIM = 512
HEADS = 8
DH = 64
OF = 256

_BN_SCALE = 1.0 / math.sqrt(1.0 + 1e-5)


_CW = 80
_BCH = 16
_CHUNKS = 256
_NBLK_IDX = _CHUNKS // _BCH
_EPAD = _CHUNKS * _CW - E // 16
_DPAD = 10040
_NPAD = 10048
_DEPTH = 4


@functools.lru_cache(maxsize=None)
def _make_segsum():
  mesh = plsc.VectorSubcoreMesh(core_axis_name="c", subcore_axis_name="s")

  @functools.partial(
      pl.kernel,
      out_type=jax.ShapeDtypeStruct((2 * N, 128), jnp.float32),
      mesh=mesh,
      scratch_types=[
          pltpu.VMEM((_BCH, _CW), jnp.int32),
          pltpu.VMEM((_BCH, _CW), jnp.int32),
          pltpu.VMEM((_BCH, _CW), jnp.int32),
          pltpu.VMEM((_BCH, _CW), jnp.int32),
          [pltpu.VMEM((_CW, 128), jnp.float32)] * _DEPTH,
          pltpu.VMEM_SHARED((_NPAD, 128), jnp.float32),
          pltpu.SemaphoreType.DMA,
          pltpu.SemaphoreType.DMA,
          [pltpu.SemaphoreType.DMA] * _DEPTH,
          [pltpu.SemaphoreType.DMA] * _DEPTH,
      ],
  )
  def segsum(gxs, srcs2, dst2, zeros, out,
             sa, da, sb, db, rows, acc,
             sem_ia, sem_ib, sem_g, sem_s):
    c = lax.axis_index("c")
    s = lax.axis_index("s")
    w8 = (c * 16 + s) * _NBLK_IDX
    d8 = s * _NBLK_IDX

    @pl.when(s < 15)
    def _():
      pltpu.sync_copy(zeros.at[pl.ds(0, 624)],
                      acc.at[pl.ds(pl.multiple_of(s * 624, 8), 624)])

    @pl.when(s == 15)
    def _():
      pltpu.sync_copy(zeros, acc.at[pl.ds(15 * 624, 688)])

    pltpu.async_copy(srcs2.at[w8], sa, sem_ia)
    pltpu.async_copy(dst2.at[d8], da, sem_ia)
    plsc.subcore_barrier()

    def inner(sv, dv):
      for b in range(_DEPTH):
        pltpu.async_copy(gxs.at[sv.at[b]], rows[b], sem_g[b])

      def step(jj, carry):
        j = _DEPTH * jj
        for b in range(_DEPTH):
          pltpu.make_async_copy(gxs.at[sv.at[j + b]], rows[b],
                                sem_g[b]).wait()
          pltpu.async_copy(rows[b], acc.at[dv.at[j + b]], sem_s[b],
                           add=True)
        for b in range(_DEPTH):
          pltpu.make_async_copy(rows[b], acc.at[dv.at[j + b]],
                                sem_s[b]).wait()

          @pl.when(j + _DEPTH + b < _BCH)
          def _():
            pltpu.async_copy(gxs.at[sv.at[j + _DEPTH + b]], rows[b],
                             sem_g[b])

        return carry

      lax.fori_loop(0, _BCH // _DEPTH, step, 0)

    def outer(bb, carry):
      b0 = 2 * bb
      pltpu.make_async_copy(srcs2.at[w8 + b0], sa, sem_ia).wait()
      pltpu.make_async_copy(dst2.at[d8 + b0], da, sem_ia).wait()
      pltpu.async_copy(srcs2.at[w8 + b0 + 1], sb, sem_ib)
      pltpu.async_copy(dst2.at[d8 + b0 + 1], db, sem_ib)
      inner(sa, da)
      pltpu.make_async_copy(srcs2.at[w8 + b0 + 1], sb, sem_ib).wait()
      pltpu.make_async_copy(dst2.at[d8 + b0 + 1], db, sem_ib).wait()

      @pl.when(bb < _NBLK_IDX // 2 - 1)
      def _():
        pltpu.async_copy(srcs2.at[w8 + b0 + 2], sa, sem_ia)
        pltpu.async_copy(dst2.at[d8 + b0 + 2], da, sem_ia)

      inner(sb, db)
      return carry

    lax.fori_loop(0, _NBLK_IDX // 2, outer, 0)
    plsc.subcore_barrier()

    @pl.when(s < 15)
    def _():
      pltpu.sync_copy(
          acc.at[pl.ds(pl.multiple_of(s * 624, 8), 624)],
          out.at[pl.ds(pl.multiple_of(c * N + s * 624, 8), 624)])

    @pl.when(s == 15)
    def _():
      pltpu.sync_copy(
          acc.at[pl.ds(15 * 624, 640)],
          out.at[pl.ds(pl.multiple_of(c * N + 15 * 624, 8), 640)])

  return segsum


_RB = 2000
_NBLK = N // _RB


def _gin_body(gx_ref, agglo_ref, agghi_ref, b_ref, w1_ref, b1_ref,
              w2_ref, b2_ref, bng_ref, bnb_ref, out_ref, pool_ref):
  i = pl.program_id(0)
  agg = jnp.concatenate([agglo_ref[...], agghi_ref[...]], axis=1)
  h = gx_ref[...] + agg
  z = jnp.maximum(
      jnp.dot(h, w1_ref[...], preferred_element_type=jnp.float32)
      + b1_ref[...], 0.0)
  z = jnp.dot(z, w2_ref[...], preferred_element_type=jnp.float32) + b2_ref[...]
  act = jnp.maximum(z * (bng_ref[...] * _BN_SCALE) + bnb_ref[...], 0.0)
  out_ref[...] = act

  b = b_ref[...]
  maxs = []
  sums = []
  for gg in range(G):
    m = b == gg
    maxs.append(jnp.max(jnp.where(m, act, -jnp.inf), axis=0).reshape(1, NH))
    sums.append(jnp.sum(jnp.where(m, act, 0.0), axis=0).reshape(1, NH))
  pmax = jnp.concatenate(maxs, axis=0)
  psum = jnp.concatenate(sums, axis=0)

  @pl.when(i == 0)
  def _():
    pool_ref[...] = jnp.concatenate(
        [jnp.full((G, NH), -jnp.inf, jnp.float32),
         jnp.zeros((G, NH), jnp.float32)], axis=1)

  cur = pool_ref[...]
  pool_ref[...] = jnp.concatenate(
      [jnp.maximum(cur[:, :NH], pmax), cur[:, NH:] + psum], axis=1)


def _gin_layer(gx, agg, batch2, w1, b1, w2, b2, bng, bnb):
  din = gx.shape[1]
  grid = (_NBLK,)
  return pl.pallas_call(
      _gin_body,
      grid=grid,
      in_specs=[
          pl.BlockSpec((_RB, din), lambda i: (i, 0)),
          pl.BlockSpec((_RB, 128), lambda i: (i, 0)),
          pl.BlockSpec((_RB, 128), lambda i: (i + _NBLK, 0)),
          pl.BlockSpec((_RB, 1), lambda i: (i, 0)),
          pl.BlockSpec((din, NH), lambda i: (0, 0)),
          pl.BlockSpec((1, NH), lambda i: (0, 0)),
          pl.BlockSpec((NH, NH), lambda i: (0, 0)),
          pl.BlockSpec((1, NH), lambda i: (0, 0)),
          pl.BlockSpec((1, NH), lambda i: (0, 0)),
          pl.BlockSpec((1, NH), lambda i: (0, 0)),
      ],
      out_specs=[
          pl.BlockSpec((_RB, NH), lambda i: (i, 0)),
          pl.BlockSpec((G, 2 * NH), lambda i: (0, 0)),
      ],
      out_shape=[
          jax.ShapeDtypeStruct((N, NH), jnp.float32),
          jax.ShapeDtypeStruct((G, 2 * NH), jnp.float32),
      ],
      compiler_params=pltpu.CompilerParams(
          dimension_semantics=("arbitrary",)),
  )(gx, agg, agg, batch2, w1, b1, w2, b2, bng, bnb)


def _ln_in(x, g, b):
  m = jnp.mean(x, axis=-1, keepdims=True)
  v = jnp.mean((x - m) ** 2, axis=-1, keepdims=True)
  return (x - m) * lax.rsqrt(v + 1e-5) * g + b


def _vit_body(pi_ref, wp_ref, bp_ref, ln1g_ref, ln1b_ref, wqkv_ref, bqkv_ref,
              wo_ref, bo_ref, ln2g_ref, ln2b_ref, wm1_ref, bm1_ref,
              wm2_ref, bm2_ref, wout_ref, bout_ref, bng_ref, bnb_ref,
              out_ref):
  xp = pi_ref[0]
  x = jnp.dot(xp, wp_ref[...], preferred_element_type=jnp.float32) + bp_ref[...]
  h = _ln_in(x, ln1g_ref[...], ln1b_ref[...])
  qkv = (jnp.dot(h, wqkv_ref[...], preferred_element_type=jnp.float32)
         + bqkv_ref[...])
  inv = 1.0 / math.sqrt(float(DH))
  outs = []
  for hd in range(HEADS):
    q = qkv[:, hd * DH:(hd + 1) * DH]
    k = qkv[:, DIM + hd * DH:DIM + (hd + 1) * DH]
    v = qkv[:, 2 * DIM + hd * DH:2 * DIM + (hd + 1) * DH]
    s = lax.dot_general(q, k, (((1,), (1,)), ((), ())),
                        preferred_element_type=jnp.float32) * inv
    s = s - jnp.max(s, axis=-1, keepdims=True)
    e = jnp.exp(s)
    a = e / jnp.sum(e, axis=-1, keepdims=True)
    outs.append(jnp.dot(a, v, preferred_element_type=jnp.float32))
  o = jnp.concatenate(outs, axis=1)
  x = x + jnp.dot(o, wo_ref[...], preferred_element_type=jnp.float32) + bo_ref[...]
  h = _ln_in(x, ln2g_ref[...], ln2b_ref[...])
  m = jnp.maximum(
      jnp.dot(h, wm1_ref[...], preferred_element_type=jnp.float32)
      + bm1_ref[...], 0.0)
  x = x + jnp.dot(m, wm2_ref[...], preferred_element_type=jnp.float32) + bm2_ref[...]
  pooled = jnp.mean(x, axis=0, keepdims=True)
  r = (jnp.dot(pooled, wout_ref[...], preferred_element_type=jnp.float32)
       + bout_ref[...])
  out_ref[...] = (r * (bng_ref[...] * _BN_SCALE) + bnb_ref[...]).reshape(
      1, 1, OF)


def _vit(pi, wp, bp, ln1g, ln1b, wqkv, bqkv, wo, bo, ln2g, ln2b,
         wm1, bm1, wm2, bm2, wout, bout, bng, bnb):
  full = lambda shape: pl.BlockSpec(shape, lambda g: tuple(0 for _ in shape))
  return pl.pallas_call(
      _vit_body,
      grid=(G,),
      in_specs=[
          pl.BlockSpec((1, P, PD), lambda g: (g, 0, 0)),
          full((PD, DIM)), full((1, DIM)), full((1, DIM)), full((1, DIM)),
          full((DIM, 3 * DIM)), full((1, 3 * DIM)),
          full((DIM, DIM)), full((1, DIM)), full((1, DIM)), full((1, DIM)),
          full((DIM, 256)), full((1, 256)), full((256, DIM)), full((1, DIM)),
          full((DIM, OF)), full((1, OF)), full((1, OF)), full((1, OF)),
      ],
      out_specs=pl.BlockSpec((1, 1, OF), lambda g: (g, 0, 0)),
      out_shape=jax.ShapeDtypeStruct((G, 1, OF), jnp.float32),
      compiler_params=pltpu.CompilerParams(
          dimension_semantics=("arbitrary",)),
  )(pi, wp, bp, ln1g, ln1b, wqkv, bqkv, wo, bo, ln2g, ln2b,
    wm1, bm1, wm2, bm2, wout, bout, bng, bnb).reshape(G, OF)


def _head_body(img_ref, p1_ref, p2_ref, p3_ref, p4_ref, bt_ref,
               l1w_ref, l1b_ref, l2w_ref, l2b_ref, gbg_ref, gbb_ref,
               f1w1_ref, f1b1_ref, f1w2_ref, f1b2_ref,
               f2w1_ref, f2b1_ref, f2w2_ref, f2b2_ref,
               hw_ref, hb_ref, out_ref):
  bt = bt_ref[...]
  cnts = []
  for gg in range(G):
    cnts.append(jnp.sum(jnp.where(bt == gg, 1.0, 0.0)).reshape(1, 1))
  cnt = jnp.maximum(jnp.concatenate(cnts, axis=0), 1.0)

  acc = None
  for p_ref in (p1_ref, p2_ref, p3_ref, p4_ref):
    p = p_ref[...]
    rep = jnp.concatenate([p[:, :NH], p[:, NH:] / cnt], axis=1)
    acc = rep if acc is None else acc + rep
  r1 = jnp.maximum(
      jnp.dot(acc, l1w_ref[...], preferred_element_type=jnp.float32)
      + l1b_ref[...], 0.0)
  go = (jnp.dot(r1, l2w_ref[...], preferred_element_type=jnp.float32)
        + l2b_ref[...]) * (gbg_ref[...] * _BN_SCALE) + gbb_ref[...]
  f = jnp.concatenate([img_ref[...], go], axis=1)
  f = jnp.maximum(jnp.dot(f, f1w1_ref[...], preferred_element_type=jnp.float32)
                  + f1b1_ref[...], 0.0)
  f = jnp.maximum(jnp.dot(f, f1w2_ref[...], preferred_element_type=jnp.float32)
                  + f1b2_ref[...], 0.0)
  f = jnp.maximum(jnp.dot(f, f2w1_ref[...], preferred_element_type=jnp.float32)
                  + f2b1_ref[...], 0.0)
  f = jnp.maximum(jnp.dot(f, f2w2_ref[...], preferred_element_type=jnp.float32)
                  + f2b2_ref[...], 0.0)
  out_ref[...] = (jnp.dot(f, hw_ref[...], preferred_element_type=jnp.float32)
                  + hb_ref[...])


def _head(img, p1, p2, p3, p4, bt, l1w, l1b, l2w, l2b, gbg, gbb,
          f1w1, f1b1, f1w2, f1b2, f2w1, f2b1, f2w2, f2b2, hw, hb):
  return pl.pallas_call(
      _head_body,
      out_shape=jax.ShapeDtypeStruct((G, hw.shape[1]), jnp.float32),
  )(img, p1, p2, p3, p4, bt, l1w, l1b, l2w, l2b, gbg, gbb,
    f1w1, f1b1, f1w2, f1b2, f2w1, f2b1, f2w2, f2b2, hw, hb)


def kernel(x, edge_index, edge_attr, patch_img, batch, vit_Wp, vit_bp,
           vit_ln1_g, vit_ln1_b, vit_Wqkv, vit_bqkv, vit_Wo, vit_bo,
           vit_ln2_g, vit_ln2_b, vit_Wm1, vit_bm1, vit_Wm2, vit_bm2,
           vit_Wout, vit_bout, vit_bn_g, vit_bn_b,
           g1_W1, g1_b1, g1_W2, g1_b2, bn1_g, bn1_b,
           g2_W1, g2_b1, g2_W2, g2_b2, bn2_g, bn2_b,
           g3_W1, g3_b1, g3_W2, g3_b2, bn3_g, bn3_b,
           g4_W1, g4_b1, g4_W2, g4_b2, bn4_g, bn4_b,
           lin1_W, lin1_b, lin2_W, lin2_b, ginbn_g, ginbn_b,
           f1_W1, f1_b1, f1_W2, f1_b2, f2_W1, f2_b1, f2_W2, f2_b2,
           head_W, head_b):
  del edge_attr
  r2 = lambda v: v.reshape(1, -1)

  src = jnp.concatenate(
      [edge_index[0].reshape(16, E // 16),
       jnp.zeros((16, _EPAD), jnp.int32)],
      axis=1).reshape(16 * _NBLK_IDX, _BCH, _CW)
  srcs2 = jnp.concatenate([src, src + N], axis=0)
  dst2 = jnp.concatenate(
      [edge_index[1].reshape(16, E // 16),
       jnp.full((16, _EPAD), _DPAD, jnp.int32)],
      axis=1).reshape(16 * _NBLK_IDX, _BCH, _CW)
  batch2 = batch.reshape(N, 1)
  bt = batch.reshape(100, 100)
  zeros128 = jnp.zeros((688, 128), jnp.float32)

  img = _vit(patch_img, vit_Wp, r2(vit_bp), r2(vit_ln1_g), r2(vit_ln1_b),
             vit_Wqkv, r2(vit_bqkv), vit_Wo, r2(vit_bo), r2(vit_ln2_g),
             r2(vit_ln2_b), vit_Wm1, r2(vit_bm1), vit_Wm2, r2(vit_bm2),
             vit_Wout, r2(vit_bout), r2(vit_bn_g), r2(vit_bn_b))

  x_pad = jnp.concatenate([x, jnp.zeros((N, 128), jnp.float32)], axis=1)
  w1s = jnp.stack([
      jnp.concatenate([g1_W1, jnp.zeros((128, NH), jnp.float32)], axis=0),
      g2_W1, g3_W1, g4_W1])
  w2s = jnp.stack([g1_W2, g2_W2, g3_W2, g4_W2])
  b1s = jnp.stack([r2(g1_b1), r2(g2_b1), r2(g3_b1), r2(g4_b1)])
  b2s = jnp.stack([r2(g1_b2), r2(g2_b2), r2(g3_b2), r2(g4_b2)])
  bgs = jnp.stack([r2(bn1_g), r2(bn2_g), r2(bn3_g), r2(bn4_g)])
  bbs = jnp.stack([r2(bn1_b), r2(bn2_b), r2(bn3_b), r2(bn4_b)])

  def gin_step(l, carry):
    gx, pools = carry
    gxs = jnp.concatenate([gx[:, :128], gx[:, 128:]], axis=0)
    agg = _make_segsum()(gxs, srcs2, dst2, zeros128)
    gxn, pool = _gin_layer(gx, agg, batch2, w1s[l], b1s[l], w2s[l], b2s[l],
                           bgs[l], bbs[l])
    pools = lax.dynamic_update_slice(pools, pool[None], (l, 0, 0))
    return gxn, pools

  _, pools = lax.fori_loop(0, 4, gin_step,
                           (x_pad, jnp.zeros((4, G, 2 * NH), jnp.float32)))

  y = _head(img, pools[0], pools[1], pools[2], pools[3], bt,
            lin1_W, r2(lin1_b), lin2_W, r2(lin2_b), r2(ginbn_g), r2(ginbn_b),
            f1_W1, r2(f1_b1), f1_W2, r2(f1_b2), f2_W1, r2(f2_b1),
            f2_W2, r2(f2_b2), head_W, r2(head_b))
  return y
